# Initial kernel scaffold; baseline (speedup 1.0000x reference)
import jax
import jax.numpy as jnp
from jax.experimental import pallas as pl


def kernel(x, edge_idx, W1, W2, W3, g1, b1, g2, b2, g3, b3):
    raise NotImplementedError("write your pallas kernel here")


# speedup vs baseline: 17.6323x; 17.6323x over previous
import functools

import jax
import jax.numpy as jnp
from jax import lax
from jax.experimental import pallas as pl
from jax.experimental.pallas import tpu as pltpu
from jax.experimental.pallas import tpu_sc as plsc

N = 10000
E = 320000
D = 128
NPAD = 10240
EPAD = 32 * 80 * 128
ROWS_T = 80
ROWS_OUT = 640
EPS = 1e-5

_mesh = lambda: plsc.VectorSubcoreMesh(core_axis_name="c", subcore_axis_name="s")


def _deg_body(ones_hbm, dst_hbm, out_hbm, dst_v, ones_v, acc):
    c = lax.axis_index("c")
    s = lax.axis_index("s")
    tid = c * 16 + s
    pltpu.sync_copy(ones_hbm.at[pl.ds(s * ROWS_OUT, ROWS_OUT)],
                    acc.at[pl.ds(s * ROWS_OUT, ROWS_OUT)])
    pltpu.sync_copy(ones_hbm.at[pl.ds(0, 128)], ones_v)
    pltpu.sync_copy(dst_hbm.at[pl.ds(tid * ROWS_T, ROWS_T)], dst_v)
    plsc.subcore_barrier()

    def body(j, carry):
        pltpu.sync_copy(ones_v, acc.at[dst_v.at[j]], add=True)
        return carry

    lax.fori_loop(0, ROWS_T, body, 0)
    plsc.subcore_barrier()
    pltpu.sync_copy(acc.at[pl.ds(s * ROWS_OUT, ROWS_OUT)],
                    out_hbm.at[c].at[pl.ds(s * ROWS_OUT, ROWS_OUT)])


_deg_call = pl.kernel(
    _deg_body,
    out_type=jax.ShapeDtypeStruct((2, NPAD, D), jnp.float32),
    mesh=_mesh(),
    scratch_types=[
        pltpu.VMEM((ROWS_T, 128), jnp.int32),
        pltpu.VMEM((128, D), jnp.float32),
        pltpu.VMEM_SHARED((NPAD, D), jnp.float32),
    ],
)


def _agg_body(u_hbm, src_hbm, dst_hbm, out_hbm, src_v, dst_v, rows_v, acc, sem):
    c = lax.axis_index("c")
    s = lax.axis_index("s")
    tid = c * 16 + s
    pltpu.sync_copy(u_hbm.at[pl.ds(s * ROWS_OUT, ROWS_OUT)],
                    acc.at[pl.ds(s * ROWS_OUT, ROWS_OUT)])
    pltpu.sync_copy(src_hbm.at[pl.ds(tid * ROWS_T, ROWS_T)], src_v)
    pltpu.sync_copy(dst_hbm.at[pl.ds(tid * ROWS_T, ROWS_T)], dst_v)
    plsc.subcore_barrier()

    def body(j, carry):
        pltpu.async_copy(u_hbm.at[src_v.at[j]], rows_v, sem).wait()
        pltpu.sync_copy(rows_v, acc.at[dst_v.at[j]], add=True)
        return carry

    lax.fori_loop(0, ROWS_T, body, 0)
    plsc.subcore_barrier()
    pltpu.sync_copy(acc.at[pl.ds(s * ROWS_OUT, ROWS_OUT)],
                    out_hbm.at[c].at[pl.ds(s * ROWS_OUT, ROWS_OUT)])


_agg_call = pl.kernel(
    _agg_body,
    out_type=jax.ShapeDtypeStruct((2, NPAD, D), jnp.float32),
    mesh=_mesh(),
    scratch_types=[
        pltpu.VMEM((ROWS_T, 128), jnp.int32),
        pltpu.VMEM((ROWS_T, 128), jnp.int32),
        pltpu.VMEM((128, D), jnp.float32),
        pltpu.VMEM_SHARED((NPAD, D), jnp.float32),
        pltpu.SemaphoreType.DMA,
    ],
)


def _tc1_body(degp_ref, x_ref, w_ref, d_ref, u_ref):
    deg = jnp.maximum(degp_ref[0, :, 0] + degp_ref[1, :, 0] - 1.0, 1.0)
    d = lax.rsqrt(deg)
    d_ref[...] = jnp.broadcast_to(d[:, None], (NPAD, D))
    h = jnp.dot(x_ref[...], w_ref[...], preferred_element_type=jnp.float32)
    u_ref[0:N, :] = d[0:N, None] * h
    u_ref[N:, :] = jnp.zeros((NPAD - N, D), jnp.float32)


def _bn(v, g, b):
    mean = jnp.mean(v, axis=0)
    var = jnp.mean((v - mean) ** 2, axis=0)
    return (v - mean) / jnp.sqrt(var + EPS) * g + b


def _tc2_body(p_ref, u_ref, d_ref, w_ref, g_ref, b_ref, y_ref, u2_ref):
    pre = d_ref[...] * (p_ref[0] + p_ref[1] - u_ref[...])
    y = jnp.maximum(_bn(pre[0:N], g_ref[...], b_ref[...]), 0.0)
    y_ref[0:N, :] = y
    y_ref[N:, :] = jnp.zeros((NPAD - N, D), jnp.float32)
    h2 = jnp.dot(y, w_ref[...], preferred_element_type=jnp.float32)
    u2_ref[0:N, :] = d_ref[0:N, :] * h2
    u2_ref[N:, :] = jnp.zeros((NPAD - N, D), jnp.float32)


def _tc3_body(p_ref, u_ref, d_ref, y1_ref, w_ref, g_ref, b_ref, u3_ref):
    pre = d_ref[...] * (p_ref[0] + p_ref[1] - u_ref[...])
    y2 = jnp.maximum(_bn(pre[0:N], g_ref[...], b_ref[...]), 0.0)
    h3 = y1_ref[0:N, :] + y2
    u3_ref[0:N, :] = d_ref[0:N, :] * jnp.dot(
        h3, w_ref[...], preferred_element_type=jnp.float32)
    u3_ref[N:, :] = jnp.zeros((NPAD - N, D), jnp.float32)


def _tc4_body(p_ref, u_ref, d_ref, g_ref, b_ref, out_ref):
    pre = d_ref[...] * (p_ref[0] + p_ref[1] - u_ref[...])
    out_ref[...] = _bn(pre[0:N], g_ref[...], b_ref[...])


def _tc_call(body, out_shapes):
    return pl.pallas_call(body, out_shape=out_shapes)


@jax.jit
def kernel(x, edge_idx, W1, W2, W3, g1, b1, g2, b2, g3, b3):
    ei = edge_idx.astype(jnp.int32)
    pad = N + (jnp.arange(EPAD - E, dtype=jnp.int32) % (NPAD - N))
    src = jnp.concatenate([ei[0], pad]).reshape(32 * ROWS_T, 128)
    dst = jnp.concatenate([ei[1], pad]).reshape(32 * ROWS_T, 128)
    ones_tbl = jnp.zeros((NPAD, D), jnp.float32).at[0:N].set(1.0)
    g1r = g1.reshape(1, D); b1r = b1.reshape(1, D)
    g2r = g2.reshape(1, D); b2r = b2.reshape(1, D)
    g3r = g3.reshape(1, D); b3r = b3.reshape(1, D)

    degp = _deg_call(ones_tbl, dst)

    d_full, u1 = _tc_call(_tc1_body, (
        jax.ShapeDtypeStruct((NPAD, D), jnp.float32),
        jax.ShapeDtypeStruct((NPAD, D), jnp.float32),
    ))(degp, x, W1)

    p1 = _agg_call(u1, src, dst)
    y1, u2 = _tc_call(_tc2_body, (
        jax.ShapeDtypeStruct((NPAD, D), jnp.float32),
        jax.ShapeDtypeStruct((NPAD, D), jnp.float32),
    ))(p1, u1, d_full, W2, g1r, b1r)

    p2 = _agg_call(u2, src, dst)
    (u3,) = _tc_call(_tc3_body, (
        jax.ShapeDtypeStruct((NPAD, D), jnp.float32),
    ))(p2, u2, d_full, y1, W3, g2r, b2r)

    p3 = _agg_call(u3, src, dst)
    (out,) = _tc_call(_tc4_body, (
        jax.ShapeDtypeStruct((N, D), jnp.float32),
    ))(p3, u3, d_full, g3r, b3r)
    return out


# speedup vs baseline: 21.7948x; 1.2361x over previous
import functools

import jax
import jax.numpy as jnp
from jax import lax
from jax.experimental import pallas as pl
from jax.experimental.pallas import tpu as pltpu
from jax.experimental.pallas import tpu_sc as plsc

N = 10000
E = 320000
D = 128
NPAD = 10240
EPAD = 32 * 80 * 128
ROWS_T = 80
ROWS_OUT = 640
EPS = 1e-5

_mesh = lambda: plsc.VectorSubcoreMesh(core_axis_name="c", subcore_axis_name="s")


def _deg_body(ones_hbm, dst_hbm, out_hbm, dst_v, ones_v, acc):
    c = lax.axis_index("c")
    s = lax.axis_index("s")
    tid = c * 16 + s
    pltpu.sync_copy(ones_hbm.at[pl.ds(s * ROWS_OUT, ROWS_OUT)],
                    acc.at[pl.ds(s * ROWS_OUT, ROWS_OUT)])
    pltpu.sync_copy(ones_hbm.at[pl.ds(0, 128)], ones_v)
    pltpu.sync_copy(dst_hbm.at[pl.ds(tid * ROWS_T, ROWS_T)], dst_v)
    plsc.subcore_barrier()

    def body(j, carry):
        pltpu.sync_copy(ones_v, acc.at[dst_v.at[j]], add=True)
        return carry

    lax.fori_loop(0, ROWS_T, body, 0)
    plsc.subcore_barrier()
    pltpu.sync_copy(acc.at[pl.ds(s * ROWS_OUT, ROWS_OUT)],
                    out_hbm.at[c].at[pl.ds(s * ROWS_OUT, ROWS_OUT)])


_deg_call = pl.kernel(
    _deg_body,
    out_type=jax.ShapeDtypeStruct((2, NPAD, D), jnp.float32),
    mesh=_mesh(),
    scratch_types=[
        pltpu.VMEM((ROWS_T, 128), jnp.int32),
        pltpu.VMEM((128, D), jnp.float32),
        pltpu.VMEM_SHARED((NPAD, D), jnp.float32),
    ],
)


NBUF = 2


ROWS_H = ROWS_T // 2


def _agg_body(u_hbm, src_hbm, dst_hbm, out_hbm, src_v, dst_v,
              rows0, rows1, acc, sem):
    c = lax.axis_index("c")
    s = lax.axis_index("s")
    tid = c * 16 + s
    rows = [rows0, rows1]
    pltpu.sync_copy(u_hbm.at[pl.ds(s * ROWS_OUT, ROWS_OUT)],
                    acc.at[pl.ds(s * ROWS_OUT, ROWS_OUT)])
    plsc.subcore_barrier()

    def body(g, carry):
        descs = [
            pltpu.async_copy(u_hbm.at[src_v.at[g * NBUF + b]], rows[b], sem)
            for b in range(NBUF)
        ]
        for b in range(NBUF):
            descs[b].wait()
            pltpu.sync_copy(rows[b], acc.at[dst_v.at[g * NBUF + b]], add=True)
        return carry

    for h in range(2):
        base = tid * ROWS_T + h * ROWS_H
        pltpu.sync_copy(src_hbm.at[pl.ds(base, ROWS_H)], src_v)
        pltpu.sync_copy(dst_hbm.at[pl.ds(base, ROWS_H)], dst_v)
        lax.fori_loop(0, ROWS_H // NBUF, body, 0)

    plsc.subcore_barrier()
    pltpu.sync_copy(acc.at[pl.ds(s * ROWS_OUT, ROWS_OUT)],
                    out_hbm.at[c].at[pl.ds(s * ROWS_OUT, ROWS_OUT)])


_agg_call = pl.kernel(
    _agg_body,
    out_type=jax.ShapeDtypeStruct((2, NPAD, D), jnp.float32),
    mesh=_mesh(),
    scratch_types=[
        pltpu.VMEM((ROWS_H, 128), jnp.int32),
        pltpu.VMEM((ROWS_H, 128), jnp.int32),
        pltpu.VMEM((128, D), jnp.float32),
        pltpu.VMEM((128, D), jnp.float32),
        pltpu.VMEM_SHARED((NPAD, D), jnp.float32),
        pltpu.SemaphoreType.DMA,
    ],
)


def _tc1_body(degp_ref, x_ref, w_ref, d_ref, u_ref):
    deg = jnp.maximum(degp_ref[0, :, 0] + degp_ref[1, :, 0] - 1.0, 1.0)
    d = lax.rsqrt(deg)
    d_ref[...] = jnp.broadcast_to(d[:, None], (NPAD, D))
    h = jnp.dot(x_ref[...], w_ref[...], preferred_element_type=jnp.float32)
    u_ref[0:N, :] = d[0:N, None] * h
    u_ref[N:, :] = jnp.zeros((NPAD - N, D), jnp.float32)


def _bn(v, g, b):
    mean = jnp.mean(v, axis=0)
    var = jnp.mean((v - mean) ** 2, axis=0)
    return (v - mean) / jnp.sqrt(var + EPS) * g + b


def _tc2_body(p_ref, u_ref, d_ref, w_ref, g_ref, b_ref, y_ref, u2_ref):
    pre = d_ref[...] * (p_ref[0] + p_ref[1] - u_ref[...])
    y = jnp.maximum(_bn(pre[0:N], g_ref[...], b_ref[...]), 0.0)
    y_ref[0:N, :] = y
    y_ref[N:, :] = jnp.zeros((NPAD - N, D), jnp.float32)
    h2 = jnp.dot(y, w_ref[...], preferred_element_type=jnp.float32)
    u2_ref[0:N, :] = d_ref[0:N, :] * h2
    u2_ref[N:, :] = jnp.zeros((NPAD - N, D), jnp.float32)


def _tc3_body(p_ref, u_ref, d_ref, y1_ref, w_ref, g_ref, b_ref, u3_ref):
    pre = d_ref[...] * (p_ref[0] + p_ref[1] - u_ref[...])
    y2 = jnp.maximum(_bn(pre[0:N], g_ref[...], b_ref[...]), 0.0)
    h3 = y1_ref[0:N, :] + y2
    u3_ref[0:N, :] = d_ref[0:N, :] * jnp.dot(
        h3, w_ref[...], preferred_element_type=jnp.float32)
    u3_ref[N:, :] = jnp.zeros((NPAD - N, D), jnp.float32)


def _tc4_body(p_ref, u_ref, d_ref, g_ref, b_ref, out_ref):
    pre = d_ref[...] * (p_ref[0] + p_ref[1] - u_ref[...])
    out_ref[...] = _bn(pre[0:N], g_ref[...], b_ref[...])


def _tc_call(body, out_shapes):
    return pl.pallas_call(body, out_shape=out_shapes)


@jax.jit
def kernel(x, edge_idx, W1, W2, W3, g1, b1, g2, b2, g3, b3):
    ei = edge_idx.astype(jnp.int32)
    pad = N + (jnp.arange(EPAD - E, dtype=jnp.int32) % (NPAD - N))
    src = jnp.concatenate([ei[0], pad]).reshape(32 * ROWS_T, 128)
    dst = jnp.concatenate([ei[1], pad]).reshape(32 * ROWS_T, 128)
    ones_tbl = jnp.zeros((NPAD, D), jnp.float32).at[0:N].set(1.0)
    g1r = g1.reshape(1, D); b1r = b1.reshape(1, D)
    g2r = g2.reshape(1, D); b2r = b2.reshape(1, D)
    g3r = g3.reshape(1, D); b3r = b3.reshape(1, D)

    degp = _deg_call(ones_tbl, dst)

    d_full, u1 = _tc_call(_tc1_body, (
        jax.ShapeDtypeStruct((NPAD, D), jnp.float32),
        jax.ShapeDtypeStruct((NPAD, D), jnp.float32),
    ))(degp, x, W1)

    p1 = _agg_call(u1, src, dst)
    y1, u2 = _tc_call(_tc2_body, (
        jax.ShapeDtypeStruct((NPAD, D), jnp.float32),
        jax.ShapeDtypeStruct((NPAD, D), jnp.float32),
    ))(p1, u1, d_full, W2, g1r, b1r)

    p2 = _agg_call(u2, src, dst)
    (u3,) = _tc_call(_tc3_body, (
        jax.ShapeDtypeStruct((NPAD, D), jnp.float32),
    ))(p2, u2, d_full, y1, W3, g2r, b2r)

    p3 = _agg_call(u3, src, dst)
    (out,) = _tc_call(_tc4_body, (
        jax.ShapeDtypeStruct((N, D), jnp.float32),
    ))(p3, u3, d_full, g3r, b3r)
    return out


# speedup vs baseline: 21.8539x; 1.0027x over previous
import functools

import jax
import jax.numpy as jnp
from jax import lax
from jax.experimental import pallas as pl
from jax.experimental.pallas import tpu as pltpu
from jax.experimental.pallas import tpu_sc as plsc

N = 10000
E = 320000
D = 128
NPAD = 10240
EPAD = 32 * 80 * 128
ROWS_T = 80
ROWS_OUT = 640
EPS = 1e-5

_mesh = lambda: plsc.VectorSubcoreMesh(core_axis_name="c", subcore_axis_name="s")


def _deg_body(ones_hbm, dst_hbm, out_hbm, dst_v, ones_v, acc, sem):
    c = lax.axis_index("c")
    s = lax.axis_index("s")
    tid = c * 16 + s
    pltpu.sync_copy(ones_hbm.at[pl.ds(s * ROWS_OUT, ROWS_OUT)],
                    acc.at[pl.ds(s * ROWS_OUT, ROWS_OUT)])
    pltpu.sync_copy(ones_hbm.at[pl.ds(0, 128)], ones_v)
    pltpu.sync_copy(dst_hbm.at[pl.ds(tid * ROWS_T, ROWS_T)], dst_v)
    plsc.subcore_barrier()

    def body(j, carry):
        pltpu.async_copy(ones_v, acc.at[dst_v.at[j]], add=True, sem=sem)
        return carry

    lax.fori_loop(0, ROWS_T, body, 0)

    def drain(j, carry):
        pltpu.make_async_copy(ones_hbm.at[pl.ds(0, 128)], ones_v, sem).wait()
        return carry

    lax.fori_loop(0, ROWS_T, drain, 0)
    plsc.subcore_barrier()
    pltpu.sync_copy(acc.at[pl.ds(s * ROWS_OUT, ROWS_OUT)],
                    out_hbm.at[c].at[pl.ds(s * ROWS_OUT, ROWS_OUT)])


_deg_call = pl.kernel(
    _deg_body,
    out_type=jax.ShapeDtypeStruct((2, NPAD, D), jnp.float32),
    mesh=_mesh(),
    scratch_types=[
        pltpu.VMEM((ROWS_T, 128), jnp.int32),
        pltpu.VMEM((128, D), jnp.float32),
        pltpu.VMEM_SHARED((NPAD, D), jnp.float32),
        pltpu.SemaphoreType.DMA,
    ],
)


NBUF = 2


ROWS_H = ROWS_T // 2


def _agg_body(u_hbm, src_hbm, dst_hbm, out_hbm, src_v, dst_v,
              rows0, rows1, acc, sem):
    c = lax.axis_index("c")
    s = lax.axis_index("s")
    tid = c * 16 + s
    rows = [rows0, rows1]
    pltpu.sync_copy(u_hbm.at[pl.ds(s * ROWS_OUT, ROWS_OUT)],
                    acc.at[pl.ds(s * ROWS_OUT, ROWS_OUT)])
    plsc.subcore_barrier()

    def body(g, carry):
        descs = [
            pltpu.async_copy(u_hbm.at[src_v.at[g * NBUF + b]], rows[b], sem)
            for b in range(NBUF)
        ]
        for b in range(NBUF):
            descs[b].wait()
            pltpu.sync_copy(rows[b], acc.at[dst_v.at[g * NBUF + b]], add=True)
        return carry

    for h in range(2):
        base = tid * ROWS_T + h * ROWS_H
        pltpu.sync_copy(src_hbm.at[pl.ds(base, ROWS_H)], src_v)
        pltpu.sync_copy(dst_hbm.at[pl.ds(base, ROWS_H)], dst_v)
        lax.fori_loop(0, ROWS_H // NBUF, body, 0)

    plsc.subcore_barrier()
    pltpu.sync_copy(acc.at[pl.ds(s * ROWS_OUT, ROWS_OUT)],
                    out_hbm.at[c].at[pl.ds(s * ROWS_OUT, ROWS_OUT)])


_agg_call = pl.kernel(
    _agg_body,
    out_type=jax.ShapeDtypeStruct((2, NPAD, D), jnp.float32),
    mesh=_mesh(),
    scratch_types=[
        pltpu.VMEM((ROWS_H, 128), jnp.int32),
        pltpu.VMEM((ROWS_H, 128), jnp.int32),
        pltpu.VMEM((128, D), jnp.float32),
        pltpu.VMEM((128, D), jnp.float32),
        pltpu.VMEM_SHARED((NPAD, D), jnp.float32),
        pltpu.SemaphoreType.DMA,
    ],
)


def _tc1_body(degp_ref, x_ref, w_ref, d_ref, u_ref):
    deg = jnp.maximum(degp_ref[0, :, 0] + degp_ref[1, :, 0] - 1.0, 1.0)
    d = lax.rsqrt(deg)
    d_ref[...] = jnp.broadcast_to(d[:, None], (NPAD, D))
    h = jnp.dot(x_ref[...], w_ref[...], preferred_element_type=jnp.float32)
    u_ref[0:N, :] = d[0:N, None] * h
    u_ref[N:, :] = jnp.zeros((NPAD - N, D), jnp.float32)


def _bn(v, g, b):
    mean = jnp.mean(v, axis=0)
    var = jnp.mean((v - mean) ** 2, axis=0)
    return (v - mean) / jnp.sqrt(var + EPS) * g + b


def _tc2_body(p_ref, u_ref, d_ref, w_ref, g_ref, b_ref, y_ref, u2_ref):
    pre = d_ref[...] * (p_ref[0] + p_ref[1] - u_ref[...])
    y = jnp.maximum(_bn(pre[0:N], g_ref[...], b_ref[...]), 0.0)
    y_ref[0:N, :] = y
    y_ref[N:, :] = jnp.zeros((NPAD - N, D), jnp.float32)
    h2 = jnp.dot(y, w_ref[...], preferred_element_type=jnp.float32)
    u2_ref[0:N, :] = d_ref[0:N, :] * h2
    u2_ref[N:, :] = jnp.zeros((NPAD - N, D), jnp.float32)


def _tc3_body(p_ref, u_ref, d_ref, y1_ref, w_ref, g_ref, b_ref, u3_ref):
    pre = d_ref[...] * (p_ref[0] + p_ref[1] - u_ref[...])
    y2 = jnp.maximum(_bn(pre[0:N], g_ref[...], b_ref[...]), 0.0)
    h3 = y1_ref[0:N, :] + y2
    u3_ref[0:N, :] = d_ref[0:N, :] * jnp.dot(
        h3, w_ref[...], preferred_element_type=jnp.float32)
    u3_ref[N:, :] = jnp.zeros((NPAD - N, D), jnp.float32)


def _tc4_body(p_ref, u_ref, d_ref, g_ref, b_ref, out_ref):
    pre = d_ref[...] * (p_ref[0] + p_ref[1] - u_ref[...])
    out_ref[...] = _bn(pre[0:N], g_ref[...], b_ref[...])


def _tc_call(body, out_shapes):
    return pl.pallas_call(body, out_shape=out_shapes)


@jax.jit
def kernel(x, edge_idx, W1, W2, W3, g1, b1, g2, b2, g3, b3):
    ei = edge_idx.astype(jnp.int32)
    pad = N + (jnp.arange(EPAD - E, dtype=jnp.int32) % (NPAD - N))
    src = jnp.concatenate([ei[0], pad]).reshape(32 * ROWS_T, 128)
    dst = jnp.concatenate([ei[1], pad]).reshape(32 * ROWS_T, 128)
    ones_tbl = jnp.zeros((NPAD, D), jnp.float32).at[0:N].set(1.0)
    g1r = g1.reshape(1, D); b1r = b1.reshape(1, D)
    g2r = g2.reshape(1, D); b2r = b2.reshape(1, D)
    g3r = g3.reshape(1, D); b3r = b3.reshape(1, D)

    degp = _deg_call(ones_tbl, dst)

    d_full, u1 = _tc_call(_tc1_body, (
        jax.ShapeDtypeStruct((NPAD, D), jnp.float32),
        jax.ShapeDtypeStruct((NPAD, D), jnp.float32),
    ))(degp, x, W1)

    p1 = _agg_call(u1, src, dst)
    y1, u2 = _tc_call(_tc2_body, (
        jax.ShapeDtypeStruct((NPAD, D), jnp.float32),
        jax.ShapeDtypeStruct((NPAD, D), jnp.float32),
    ))(p1, u1, d_full, W2, g1r, b1r)

    p2 = _agg_call(u2, src, dst)
    (u3,) = _tc_call(_tc3_body, (
        jax.ShapeDtypeStruct((NPAD, D), jnp.float32),
    ))(p2, u2, d_full, y1, W3, g2r, b2r)

    p3 = _agg_call(u3, src, dst)
    (out,) = _tc_call(_tc4_body, (
        jax.ShapeDtypeStruct((N, D), jnp.float32),
    ))(p3, u3, d_full, g3r, b3r)
    return out


# speedup vs baseline: 26.0482x; 1.1919x over previous
import functools

import jax
import jax.numpy as jnp
from jax import lax
from jax.experimental import pallas as pl
from jax.experimental.pallas import tpu as pltpu
from jax.experimental.pallas import tpu_sc as plsc

N = 10000
E = 320000
D = 128
NPAD = 10240
EPAD = 32 * 160 * 64
CH = 64
CPT = 160
CPQ = 40
ROWS_OUT = 640
EPS = 1e-5

_mesh = lambda: plsc.VectorSubcoreMesh(core_axis_name="c", subcore_axis_name="s")


def _deg_body(ones_hbm, dst_hbm, out_hbm, dst_v, ones_v, acc, sem):
    c = lax.axis_index("c")
    s = lax.axis_index("s")
    tid = c * 16 + s
    pltpu.sync_copy(ones_hbm.at[pl.ds(s * ROWS_OUT, ROWS_OUT)],
                    acc.at[pl.ds(s * ROWS_OUT, ROWS_OUT)])
    pltpu.sync_copy(ones_hbm.at[pl.ds(0, CH)], ones_v)
    pltpu.sync_copy(dst_hbm.at[pl.ds(tid * CPT, CPT)], dst_v)
    plsc.subcore_barrier()

    def body(j, carry):
        pltpu.async_copy(ones_v, acc.at[dst_v.at[j]], add=True, sem=sem)
        return carry

    lax.fori_loop(0, CPT, body, 0)

    def drain(j, carry):
        pltpu.make_async_copy(ones_hbm.at[pl.ds(0, CH)], ones_v, sem).wait()
        return carry

    lax.fori_loop(0, CPT, drain, 0)
    plsc.subcore_barrier()
    pltpu.sync_copy(acc.at[pl.ds(s * ROWS_OUT, ROWS_OUT)],
                    out_hbm.at[c].at[pl.ds(s * ROWS_OUT, ROWS_OUT)])


_deg_call = pl.kernel(
    _deg_body,
    out_type=jax.ShapeDtypeStruct((2, NPAD, D), jnp.float32),
    mesh=_mesh(),
    scratch_types=[
        pltpu.VMEM((CPT, CH), jnp.int32),
        pltpu.VMEM((CH, D), jnp.float32),
        pltpu.VMEM_SHARED((NPAD, D), jnp.float32),
        pltpu.SemaphoreType.DMA,
    ],
)


NBUF = 4


def _agg_body(u_hbm, src_hbm, dst_hbm, out_hbm, src_v, dst_v,
              rows0, rows1, rows2, rows3, acc, semg, sems):
    c = lax.axis_index("c")
    s = lax.axis_index("s")
    tid = c * 16 + s
    rows = [rows0, rows1, rows2, rows3]
    pltpu.sync_copy(u_hbm.at[pl.ds(s * ROWS_OUT, ROWS_OUT)],
                    acc.at[pl.ds(s * ROWS_OUT, ROWS_OUT)])
    plsc.subcore_barrier()

    def gather(j, slot):
        pltpu.async_copy(u_hbm.at[src_v.at[j]], slot, semg)

    def wait_gather(j, slot):
        pltpu.make_async_copy(u_hbm.at[src_v.at[j]], slot, semg).wait()

    def drain_scatter(slot):
        pltpu.make_async_copy(u_hbm.at[src_v.at[0]], slot, sems).wait()

    NG = CPQ // NBUF

    def body(g, carry):
        for b in range(NBUF):
            if b == 0:
                @pl.when(g > 0)
                def _():
                    drain_scatter(rows[NBUF - 1])
            else:
                drain_scatter(rows[b - 1])

            nxt_ok = (g < NG - 1) if b > 0 else True
            if nxt_ok is True:
                gather(g * NBUF + b + NBUF - 1, rows[(b + NBUF - 1) % NBUF])
            else:
                @pl.when(g < NG - 1)
                def _(b=b):
                    gather(g * NBUF + b + NBUF - 1, rows[(b + NBUF - 1) % NBUF])

            wait_gather(g * NBUF + b, rows[b])
            pltpu.async_copy(rows[b], acc.at[dst_v.at[g * NBUF + b]], sems,
                             add=True)
        return carry

    for h in range(4):
        base = tid * CPT + h * CPQ
        pltpu.sync_copy(src_hbm.at[pl.ds(base, CPQ)], src_v)
        pltpu.sync_copy(dst_hbm.at[pl.ds(base, CPQ)], dst_v)
        for b in range(NBUF - 1):
            gather(b, rows[b])
        lax.fori_loop(0, NG, body, 0)
        drain_scatter(rows[0])

    plsc.subcore_barrier()
    pltpu.sync_copy(acc.at[pl.ds(s * ROWS_OUT, ROWS_OUT)],
                    out_hbm.at[c].at[pl.ds(s * ROWS_OUT, ROWS_OUT)])


_agg_call = pl.kernel(
    _agg_body,
    out_type=jax.ShapeDtypeStruct((2, NPAD, D), jnp.float32),
    mesh=_mesh(),
    scratch_types=[
        pltpu.VMEM((CPQ, CH), jnp.int32),
        pltpu.VMEM((CPQ, CH), jnp.int32),
        pltpu.VMEM((CH, D), jnp.float32),
        pltpu.VMEM((CH, D), jnp.float32),
        pltpu.VMEM((CH, D), jnp.float32),
        pltpu.VMEM((CH, D), jnp.float32),
        pltpu.VMEM_SHARED((NPAD, D), jnp.float32),
        pltpu.SemaphoreType.DMA,
        pltpu.SemaphoreType.DMA,
    ],
)


def _tc1_body(degp_ref, x_ref, w_ref, d_ref, u_ref):
    deg = jnp.maximum(degp_ref[0, :, 0] + degp_ref[1, :, 0] - 1.0, 1.0)
    d = lax.rsqrt(deg)
    d_ref[...] = jnp.broadcast_to(d[:, None], (NPAD, D))
    h = jnp.dot(x_ref[...], w_ref[...], preferred_element_type=jnp.float32)
    u_ref[0:N, :] = d[0:N, None] * h
    u_ref[N:, :] = jnp.zeros((NPAD - N, D), jnp.float32)


def _bn(v, g, b):
    mean = jnp.mean(v, axis=0)
    var = jnp.mean((v - mean) ** 2, axis=0)
    return (v - mean) / jnp.sqrt(var + EPS) * g + b


def _tc2_body(p_ref, u_ref, d_ref, w_ref, g_ref, b_ref, y_ref, u2_ref):
    pre = d_ref[...] * (p_ref[0] + p_ref[1] - u_ref[...])
    y = jnp.maximum(_bn(pre[0:N], g_ref[...], b_ref[...]), 0.0)
    y_ref[0:N, :] = y
    y_ref[N:, :] = jnp.zeros((NPAD - N, D), jnp.float32)
    h2 = jnp.dot(y, w_ref[...], preferred_element_type=jnp.float32)
    u2_ref[0:N, :] = d_ref[0:N, :] * h2
    u2_ref[N:, :] = jnp.zeros((NPAD - N, D), jnp.float32)


def _tc3_body(p_ref, u_ref, d_ref, y1_ref, w_ref, g_ref, b_ref, u3_ref):
    pre = d_ref[...] * (p_ref[0] + p_ref[1] - u_ref[...])
    y2 = jnp.maximum(_bn(pre[0:N], g_ref[...], b_ref[...]), 0.0)
    h3 = y1_ref[0:N, :] + y2
    u3_ref[0:N, :] = d_ref[0:N, :] * jnp.dot(
        h3, w_ref[...], preferred_element_type=jnp.float32)
    u3_ref[N:, :] = jnp.zeros((NPAD - N, D), jnp.float32)


def _tc4_body(p_ref, u_ref, d_ref, g_ref, b_ref, out_ref):
    pre = d_ref[...] * (p_ref[0] + p_ref[1] - u_ref[...])
    out_ref[...] = _bn(pre[0:N], g_ref[...], b_ref[...])


def _tc_call(body, out_shapes):
    return pl.pallas_call(body, out_shape=out_shapes)


@jax.jit
def kernel(x, edge_idx, W1, W2, W3, g1, b1, g2, b2, g3, b3):
    ei = edge_idx.astype(jnp.int32)
    pad = N + (jnp.arange(EPAD - E, dtype=jnp.int32) % (NPAD - N))
    src = jnp.concatenate([ei[0], pad]).reshape(32 * CPT, CH)
    dst = jnp.concatenate([ei[1], pad]).reshape(32 * CPT, CH)
    ones_tbl = jnp.zeros((NPAD, D), jnp.float32).at[0:N].set(1.0)
    g1r = g1.reshape(1, D); b1r = b1.reshape(1, D)
    g2r = g2.reshape(1, D); b2r = b2.reshape(1, D)
    g3r = g3.reshape(1, D); b3r = b3.reshape(1, D)

    degp = _deg_call(ones_tbl, dst)

    d_full, u1 = _tc_call(_tc1_body, (
        jax.ShapeDtypeStruct((NPAD, D), jnp.float32),
        jax.ShapeDtypeStruct((NPAD, D), jnp.float32),
    ))(degp, x, W1)

    p1 = _agg_call(u1, src, dst)
    y1, u2 = _tc_call(_tc2_body, (
        jax.ShapeDtypeStruct((NPAD, D), jnp.float32),
        jax.ShapeDtypeStruct((NPAD, D), jnp.float32),
    ))(p1, u1, d_full, W2, g1r, b1r)

    p2 = _agg_call(u2, src, dst)
    (u3,) = _tc_call(_tc3_body, (
        jax.ShapeDtypeStruct((NPAD, D), jnp.float32),
    ))(p2, u2, d_full, y1, W3, g2r, b2r)

    p3 = _agg_call(u3, src, dst)
    (out,) = _tc_call(_tc4_body, (
        jax.ShapeDtypeStruct((N, D), jnp.float32),
    ))(p3, u3, d_full, g3r, b3r)
    return out


# speedup vs baseline: 30.5393x; 1.1724x over previous
import functools

import jax
import jax.numpy as jnp
from jax import lax
from jax.experimental import pallas as pl
from jax.experimental.pallas import tpu as pltpu
from jax.experimental.pallas import tpu_sc as plsc

N = 10000
E = 320000
D = 128
NPAD = 10240
EPAD = 32 * 160 * 64
CH = 64
CPT = 160
CPQ = 40
ROWS_OUT = 640
EPS = 1e-5

_mesh = lambda: plsc.VectorSubcoreMesh(core_axis_name="c", subcore_axis_name="s")


def _deg_body(ones_hbm, dst_hbm, out_hbm, dst_v, ones_v, acc, sem):
    c = lax.axis_index("c")
    s = lax.axis_index("s")
    tid = c * 16 + s
    pltpu.sync_copy(ones_hbm.at[pl.ds(s * ROWS_OUT, ROWS_OUT)],
                    acc.at[pl.ds(s * ROWS_OUT, ROWS_OUT)])
    pltpu.sync_copy(ones_hbm.at[pl.ds(0, CH)], ones_v)
    pltpu.sync_copy(dst_hbm.at[pl.ds(tid * CPT, CPT)], dst_v)
    plsc.subcore_barrier()

    def body(j, carry):
        pltpu.async_copy(ones_v, acc.at[dst_v.at[j]], add=True, sem=sem)
        return carry

    lax.fori_loop(0, CPT, body, 0)

    def drain(j, carry):
        pltpu.make_async_copy(ones_hbm.at[pl.ds(0, CH)], ones_v, sem).wait()
        return carry

    lax.fori_loop(0, CPT, drain, 0)
    plsc.subcore_barrier()
    pltpu.sync_copy(acc.at[pl.ds(s * ROWS_OUT, ROWS_OUT)],
                    out_hbm.at[c].at[pl.ds(s * ROWS_OUT, ROWS_OUT)])


_deg_call = pl.kernel(
    _deg_body,
    out_type=jax.ShapeDtypeStruct((2, NPAD), jnp.float32),
    mesh=_mesh(),
    scratch_types=[
        pltpu.VMEM((CPT, CH), jnp.int32),
        pltpu.VMEM((CH,), jnp.float32),
        pltpu.VMEM_SHARED((NPAD,), jnp.float32),
        pltpu.SemaphoreType.DMA,
    ],
)


NBUF = 4


def _agg_body(u_hbm, src_hbm, dst_hbm, out_hbm, src_v, dst_v,
              rows0, rows1, rows2, rows3, acc, semg, sems):
    c = lax.axis_index("c")
    s = lax.axis_index("s")
    tid = c * 16 + s
    rows = [rows0, rows1, rows2, rows3]
    pltpu.sync_copy(u_hbm.at[pl.ds(s * ROWS_OUT, ROWS_OUT)],
                    acc.at[pl.ds(s * ROWS_OUT, ROWS_OUT)])
    plsc.subcore_barrier()

    def gather(j, slot):
        pltpu.async_copy(u_hbm.at[src_v.at[j]], slot, semg)

    def wait_gather(j, slot):
        pltpu.make_async_copy(u_hbm.at[src_v.at[j]], slot, semg).wait()

    def drain_scatter(slot):
        pltpu.make_async_copy(u_hbm.at[src_v.at[0]], slot, sems).wait()

    NG = CPQ // NBUF

    def body(g, carry):
        for b in range(NBUF):
            if b == 0:
                @pl.when(g > 0)
                def _():
                    drain_scatter(rows[NBUF - 1])
            else:
                drain_scatter(rows[b - 1])

            nxt_ok = (g < NG - 1) if b > 0 else True
            if nxt_ok is True:
                gather(g * NBUF + b + NBUF - 1, rows[(b + NBUF - 1) % NBUF])
            else:
                @pl.when(g < NG - 1)
                def _(b=b):
                    gather(g * NBUF + b + NBUF - 1, rows[(b + NBUF - 1) % NBUF])

            wait_gather(g * NBUF + b, rows[b])
            pltpu.async_copy(rows[b], acc.at[dst_v.at[g * NBUF + b]], sems,
                             add=True)
        return carry

    for h in range(4):
        base = tid * CPT + h * CPQ
        pltpu.sync_copy(src_hbm.at[pl.ds(base, CPQ)], src_v)
        pltpu.sync_copy(dst_hbm.at[pl.ds(base, CPQ)], dst_v)
        for b in range(NBUF - 1):
            gather(b, rows[b])
        lax.fori_loop(0, NG, body, 0)
        drain_scatter(rows[0])

    plsc.subcore_barrier()
    pltpu.sync_copy(acc.at[pl.ds(s * ROWS_OUT, ROWS_OUT)],
                    out_hbm.at[c].at[pl.ds(s * ROWS_OUT, ROWS_OUT)])


_agg_call = pl.kernel(
    _agg_body,
    out_type=jax.ShapeDtypeStruct((2, NPAD, D), jnp.float32),
    mesh=_mesh(),
    scratch_types=[
        pltpu.VMEM((CPQ, CH), jnp.int32),
        pltpu.VMEM((CPQ, CH), jnp.int32),
        pltpu.VMEM((CH, D), jnp.float32),
        pltpu.VMEM((CH, D), jnp.float32),
        pltpu.VMEM((CH, D), jnp.float32),
        pltpu.VMEM((CH, D), jnp.float32),
        pltpu.VMEM_SHARED((NPAD, D), jnp.float32),
        pltpu.SemaphoreType.DMA,
        pltpu.SemaphoreType.DMA,
    ],
)


def _tc0_body(x_ref, w_ref, h_ref):
    h_ref[...] = jnp.dot(x_ref[...], w_ref[...],
                         preferred_element_type=jnp.float32)


def _tc1_body(degp_ref, h_ref, d_ref, u_ref):
    deg = jnp.maximum(degp_ref[0, :] + degp_ref[1, :] - 1.0, 1.0)
    d = lax.rsqrt(deg)
    d_ref[...] = jnp.broadcast_to(d[:, None], (NPAD, D))
    u_ref[0:N, :] = d[0:N, None] * h_ref[...]
    u_ref[N:, :] = jnp.zeros((NPAD - N, D), jnp.float32)


def _bn(v, g, b):
    mean = jnp.mean(v, axis=0)
    var = jnp.mean((v - mean) ** 2, axis=0)
    return (v - mean) / jnp.sqrt(var + EPS) * g + b


def _tc2_body(p_ref, u_ref, d_ref, w_ref, g_ref, b_ref, y_ref, u2_ref):
    pre = d_ref[...] * (p_ref[0] + p_ref[1] - u_ref[...])
    y = jnp.maximum(_bn(pre[0:N], g_ref[...], b_ref[...]), 0.0)
    y_ref[0:N, :] = y
    y_ref[N:, :] = jnp.zeros((NPAD - N, D), jnp.float32)
    h2 = jnp.dot(y, w_ref[...], preferred_element_type=jnp.float32)
    u2_ref[0:N, :] = d_ref[0:N, :] * h2
    u2_ref[N:, :] = jnp.zeros((NPAD - N, D), jnp.float32)


def _tc3_body(p_ref, u_ref, d_ref, y1_ref, w_ref, g_ref, b_ref, u3_ref):
    pre = d_ref[...] * (p_ref[0] + p_ref[1] - u_ref[...])
    y2 = jnp.maximum(_bn(pre[0:N], g_ref[...], b_ref[...]), 0.0)
    h3 = y1_ref[0:N, :] + y2
    u3_ref[0:N, :] = d_ref[0:N, :] * jnp.dot(
        h3, w_ref[...], preferred_element_type=jnp.float32)
    u3_ref[N:, :] = jnp.zeros((NPAD - N, D), jnp.float32)


def _tc4_body(p_ref, u_ref, d_ref, g_ref, b_ref, out_ref):
    pre = d_ref[...] * (p_ref[0] + p_ref[1] - u_ref[...])
    out_ref[...] = _bn(pre[0:N], g_ref[...], b_ref[...])


def _tc_call(body, out_shapes):
    return pl.pallas_call(body, out_shape=out_shapes)


@jax.jit
def kernel(x, edge_idx, W1, W2, W3, g1, b1, g2, b2, g3, b3):
    ei = edge_idx.astype(jnp.int32)
    pad = N + (jnp.arange(EPAD - E, dtype=jnp.int32) % (NPAD - N))
    src = jnp.concatenate([ei[0], pad]).reshape(32 * CPT, CH)
    dst = jnp.concatenate([ei[1], pad]).reshape(32 * CPT, CH)
    ones_tbl = jnp.zeros((NPAD,), jnp.float32).at[0:N].set(1.0)
    g1r = g1.reshape(1, D); b1r = b1.reshape(1, D)
    g2r = g2.reshape(1, D); b2r = b2.reshape(1, D)
    g3r = g3.reshape(1, D); b3r = b3.reshape(1, D)

    degp = _deg_call(ones_tbl, dst)
    (h1,) = _tc_call(_tc0_body, (
        jax.ShapeDtypeStruct((N, D), jnp.float32),
    ))(x, W1)

    d_full, u1 = _tc_call(_tc1_body, (
        jax.ShapeDtypeStruct((NPAD, D), jnp.float32),
        jax.ShapeDtypeStruct((NPAD, D), jnp.float32),
    ))(degp, h1)

    p1 = _agg_call(u1, src, dst)
    y1, u2 = _tc_call(_tc2_body, (
        jax.ShapeDtypeStruct((NPAD, D), jnp.float32),
        jax.ShapeDtypeStruct((NPAD, D), jnp.float32),
    ))(p1, u1, d_full, W2, g1r, b1r)

    p2 = _agg_call(u2, src, dst)
    (u3,) = _tc_call(_tc3_body, (
        jax.ShapeDtypeStruct((NPAD, D), jnp.float32),
    ))(p2, u2, d_full, y1, W3, g2r, b2r)

    p3 = _agg_call(u3, src, dst)
    (out,) = _tc_call(_tc4_body, (
        jax.ShapeDtypeStruct((N, D), jnp.float32),
    ))(p3, u3, d_full, g3r, b3r)
    return out


# speedup vs baseline: 30.7322x; 1.0063x over previous
import functools

import jax
import jax.numpy as jnp
from jax import lax
from jax.experimental import pallas as pl
from jax.experimental.pallas import tpu as pltpu
from jax.experimental.pallas import tpu_sc as plsc

N = 10000
E = 320000
D = 128
NPAD = 10240
EPAD = 32 * 160 * 64
CH = 64
CPT = 160
CPQ = 40
ROWS_OUT = 640
EPS = 1e-5

_mesh = lambda: plsc.VectorSubcoreMesh(core_axis_name="c", subcore_axis_name="s")


def _deg_body(ones_hbm, dst_hbm, out_hbm, dst_v, ones_v, acc, sem):
    c = lax.axis_index("c")
    s = lax.axis_index("s")
    tid = c * 16 + s
    pltpu.sync_copy(ones_hbm.at[pl.ds(s * ROWS_OUT, ROWS_OUT)],
                    acc.at[pl.ds(s * ROWS_OUT, ROWS_OUT)])
    pltpu.sync_copy(ones_hbm.at[pl.ds(0, CH)], ones_v)
    pltpu.sync_copy(dst_hbm.at[pl.ds(tid * CPT, CPT)], dst_v)
    plsc.subcore_barrier()

    def body(j, carry):
        pltpu.async_copy(ones_v, acc.at[dst_v.at[j]], add=True, sem=sem)
        return carry

    lax.fori_loop(0, CPT, body, 0)

    def drain(j, carry):
        pltpu.make_async_copy(ones_hbm.at[pl.ds(0, CH)], ones_v, sem).wait()
        return carry

    lax.fori_loop(0, CPT, drain, 0)
    plsc.subcore_barrier()
    pltpu.sync_copy(acc.at[pl.ds(s * ROWS_OUT, ROWS_OUT)],
                    out_hbm.at[c].at[pl.ds(s * ROWS_OUT, ROWS_OUT)])


_deg_call = pl.kernel(
    _deg_body,
    out_type=jax.ShapeDtypeStruct((2, NPAD), jnp.float32),
    mesh=_mesh(),
    scratch_types=[
        pltpu.VMEM((CPT, CH), jnp.int32),
        pltpu.VMEM((CH,), jnp.float32),
        pltpu.VMEM_SHARED((NPAD,), jnp.float32),
        pltpu.SemaphoreType.DMA,
    ],
)


NBUF = 4


def _agg_body(u_hbm, src_hbm, dst_hbm, out_hbm, src_v, dst_v,
              rows0, rows1, rows2, rows3, acc, semg, sems):
    c = lax.axis_index("c")
    s = lax.axis_index("s")
    tid = c * 16 + s
    rows = [rows0, rows1, rows2, rows3]
    pltpu.sync_copy(u_hbm.at[pl.ds(s * ROWS_OUT, ROWS_OUT)],
                    acc.at[pl.ds(s * ROWS_OUT, ROWS_OUT)])
    plsc.subcore_barrier()

    def gather(j, slot):
        pltpu.async_copy(u_hbm.at[src_v.at[j]], slot, semg)

    def wait_gather(j, slot):
        pltpu.make_async_copy(u_hbm.at[src_v.at[j]], slot, semg).wait()

    def drain_scatter(slot):
        pltpu.make_async_copy(u_hbm.at[src_v.at[0]], slot, sems).wait()

    NG = CPQ // NBUF

    LOOK = 2

    def body(g, carry):
        for b in range(NBUF):
            j = g * NBUF + b
            if b < LOOK:
                @pl.when(g > 0)
                def _(b=b):
                    drain_scatter(rows[(b - LOOK) % NBUF])
            else:
                drain_scatter(rows[b - LOOK])

            if b < LOOK:
                gather(j + LOOK, rows[(b + LOOK) % NBUF])
            else:
                @pl.when(g < NG - 1)
                def _(j=j, b=b):
                    gather(j + LOOK, rows[(b + LOOK) % NBUF])

            wait_gather(j, rows[b])
            pltpu.async_copy(rows[b], acc.at[dst_v.at[j]], sems, add=True)
        return carry

    for h in range(4):
        base = tid * CPT + h * CPQ
        pltpu.sync_copy(src_hbm.at[pl.ds(base, CPQ)], src_v)
        pltpu.sync_copy(dst_hbm.at[pl.ds(base, CPQ)], dst_v)
        for b in range(LOOK):
            gather(b, rows[b])
        lax.fori_loop(0, NG, body, 0)
        for b in range(LOOK):
            drain_scatter(rows[b])

    plsc.subcore_barrier()
    pltpu.sync_copy(acc.at[pl.ds(s * ROWS_OUT, ROWS_OUT)],
                    out_hbm.at[c].at[pl.ds(s * ROWS_OUT, ROWS_OUT)])


_agg_call = pl.kernel(
    _agg_body,
    out_type=jax.ShapeDtypeStruct((2, NPAD, D), jnp.float32),
    mesh=_mesh(),
    scratch_types=[
        pltpu.VMEM((CPQ, CH), jnp.int32),
        pltpu.VMEM((CPQ, CH), jnp.int32),
        pltpu.VMEM((CH, D), jnp.float32),
        pltpu.VMEM((CH, D), jnp.float32),
        pltpu.VMEM((CH, D), jnp.float32),
        pltpu.VMEM((CH, D), jnp.float32),
        pltpu.VMEM_SHARED((NPAD, D), jnp.float32),
        pltpu.SemaphoreType.DMA,
        pltpu.SemaphoreType.DMA,
    ],
)


def _tc0_body(x_ref, w_ref, h_ref):
    h_ref[...] = jnp.dot(x_ref[...], w_ref[...],
                         preferred_element_type=jnp.float32)


def _tc1_body(degp_ref, h_ref, d_ref, u_ref):
    deg = jnp.maximum(degp_ref[0, :] + degp_ref[1, :] - 1.0, 1.0)
    d = lax.rsqrt(deg)
    d_ref[...] = jnp.broadcast_to(d[:, None], (NPAD, D))
    u_ref[0:N, :] = d[0:N, None] * h_ref[...]
    u_ref[N:, :] = jnp.zeros((NPAD - N, D), jnp.float32)


def _bn(v, g, b):
    mean = jnp.mean(v, axis=0)
    var = jnp.mean((v - mean) ** 2, axis=0)
    return (v - mean) / jnp.sqrt(var + EPS) * g + b


def _tc2_body(p_ref, u_ref, d_ref, w_ref, g_ref, b_ref, y_ref, u2_ref):
    pre = d_ref[...] * (p_ref[0] + p_ref[1] - u_ref[...])
    y = jnp.maximum(_bn(pre[0:N], g_ref[...], b_ref[...]), 0.0)
    y_ref[0:N, :] = y
    y_ref[N:, :] = jnp.zeros((NPAD - N, D), jnp.float32)
    h2 = jnp.dot(y, w_ref[...], preferred_element_type=jnp.float32)
    u2_ref[0:N, :] = d_ref[0:N, :] * h2
    u2_ref[N:, :] = jnp.zeros((NPAD - N, D), jnp.float32)


def _tc3_body(p_ref, u_ref, d_ref, y1_ref, w_ref, g_ref, b_ref, u3_ref):
    pre = d_ref[...] * (p_ref[0] + p_ref[1] - u_ref[...])
    y2 = jnp.maximum(_bn(pre[0:N], g_ref[...], b_ref[...]), 0.0)
    h3 = y1_ref[0:N, :] + y2
    u3_ref[0:N, :] = d_ref[0:N, :] * jnp.dot(
        h3, w_ref[...], preferred_element_type=jnp.float32)
    u3_ref[N:, :] = jnp.zeros((NPAD - N, D), jnp.float32)


def _tc4_body(p_ref, u_ref, d_ref, g_ref, b_ref, out_ref):
    pre = d_ref[...] * (p_ref[0] + p_ref[1] - u_ref[...])
    out_ref[...] = _bn(pre[0:N], g_ref[...], b_ref[...])


def _tc_call(body, out_shapes):
    return pl.pallas_call(body, out_shape=out_shapes)


@jax.jit
def kernel(x, edge_idx, W1, W2, W3, g1, b1, g2, b2, g3, b3):
    ei = edge_idx.astype(jnp.int32)
    pad = N + (jnp.arange(EPAD - E, dtype=jnp.int32) % (NPAD - N))
    src = jnp.concatenate([ei[0], pad]).reshape(32 * CPT, CH)
    dst = jnp.concatenate([ei[1], pad]).reshape(32 * CPT, CH)
    ones_tbl = jnp.zeros((NPAD,), jnp.float32).at[0:N].set(1.0)
    g1r = g1.reshape(1, D); b1r = b1.reshape(1, D)
    g2r = g2.reshape(1, D); b2r = b2.reshape(1, D)
    g3r = g3.reshape(1, D); b3r = b3.reshape(1, D)

    degp = _deg_call(ones_tbl, dst)
    (h1,) = _tc_call(_tc0_body, (
        jax.ShapeDtypeStruct((N, D), jnp.float32),
    ))(x, W1)

    d_full, u1 = _tc_call(_tc1_body, (
        jax.ShapeDtypeStruct((NPAD, D), jnp.float32),
        jax.ShapeDtypeStruct((NPAD, D), jnp.float32),
    ))(degp, h1)

    p1 = _agg_call(u1, src, dst)
    y1, u2 = _tc_call(_tc2_body, (
        jax.ShapeDtypeStruct((NPAD, D), jnp.float32),
        jax.ShapeDtypeStruct((NPAD, D), jnp.float32),
    ))(p1, u1, d_full, W2, g1r, b1r)

    p2 = _agg_call(u2, src, dst)
    (u3,) = _tc_call(_tc3_body, (
        jax.ShapeDtypeStruct((NPAD, D), jnp.float32),
    ))(p2, u2, d_full, y1, W3, g2r, b2r)

    p3 = _agg_call(u3, src, dst)
    (out,) = _tc_call(_tc4_body, (
        jax.ShapeDtypeStruct((N, D), jnp.float32),
    ))(p3, u3, d_full, g3r, b3r)
    return out
